# baseline (speedup 1.0000x reference)
import functools

import jax
import jax.numpy as jnp
from jax.experimental import pallas as pl
from jax.experimental.pallas import tpu as pltpu

_SIGMA = 0.4
_SAMPLE = 10000
_EPS = 1e-10

_NP = 10240
_BR = 128
_SB = 32
_BC = 128
_M = _NP // _BR
_PAD = 1e4
_C = -0.5 * 1.4426950408889634


def _fold8(t):
    return jnp.sum(t.reshape(_SB // 8, 8, _BC), axis=0)


def _rowsum_kernel(a_row_ref, b_row_ref, a_col_ref, b_col_ref,
                   ra_ref, rb_ref, rab_ref, ca_ref, cb_ref, cab_ref):
    rb_idx = pl.program_id(1)

    @pl.when(rb_idx == 0)
    def _():
        ca_ref[...] = jnp.zeros_like(ca_ref)
        cb_ref[...] = jnp.zeros_like(cb_ref)
        cab_ref[...] = jnp.zeros_like(cab_ref)

    diag = pl.multiple_of(rb_idx * _BC, _BC)
    for s in range(_BR // _SB):
        ars = a_row_ref[0, s * _SB:(s + 1) * _SB]
        brs = b_row_ref[0, s * _SB:(s + 1) * _SB]

        ac = a_col_ref[0, 0:1, pl.ds(diag, _BC)]
        bc = b_col_ref[0, 0:1, pl.ds(diag, _BC)]
        da = ac - ars
        db = bc - brs
        ka = jnp.exp2(da * da * _C)
        kb = jnp.exp2(db * db * _C)
        acc0 = (ka, kb, ka * kb)

        def chunk(c, acc):
            col = pl.multiple_of(c * _BC, _BC)
            ac = a_col_ref[0, 0:1, pl.ds(col, _BC)]
            bc = b_col_ref[0, 0:1, pl.ds(col, _BC)]
            da = ac - ars
            db = bc - brs
            ka = jnp.exp2(da * da * _C)
            kb = jnp.exp2(db * db * _C)
            kab = ka * kb
            ca_ref[0, :, pl.ds(col, _BC)] += _fold8(ka)
            cb_ref[0, :, pl.ds(col, _BC)] += _fold8(kb)
            cab_ref[0, :, pl.ds(col, _BC)] += _fold8(kab)
            return (acc[0] + ka, acc[1] + kb, acc[2] + kab)

        acc_a, acc_b, acc_ab = jax.lax.fori_loop(rb_idx + 1, _M, chunk, acc0)
        sl = slice(s * _SB, (s + 1) * _SB)
        ra_ref[0, sl, :] = jnp.sum(acc_a, axis=1, keepdims=True)
        rb_ref[0, sl, :] = jnp.sum(acc_b, axis=1, keepdims=True)
        rab_ref[0, sl, :] = jnp.sum(acc_ab, axis=1, keepdims=True)


def _mi_kernel(ra_ref, rb_ref, rab_ref, ca_ref, cb_ref, cab_ref, o_ref):
    rows = jax.lax.broadcasted_iota(jnp.int32, (_NP // 128, 128), 0)
    lanes = jax.lax.broadcasted_iota(jnp.int32, (_NP // 128, 128), 1)
    mask = (rows * 128 + lanes) < _SAMPLE
    inv_n = jnp.float32(1.0 / _SAMPLE)

    def total(row_ref, col_ref, p):
        s = row_ref[p]
        c = col_ref[p]
        for i in range(8):
            s = s + c[i]
        return s

    def one_pair(p):
        rm_a = jnp.where(mask, total(ra_ref, ca_ref, p) * inv_n, 0.0)
        rm_b = jnp.where(mask, total(rb_ref, cb_ref, p) * inv_n, 0.0)
        rm_ab = jnp.where(mask, total(rab_ref, cab_ref, p) * inv_n, 0.0)
        p_a = rm_a / (jnp.sum(rm_a) + _EPS)
        p_b = rm_b / (jnp.sum(rm_b) + _EPS)
        p_ab = rm_ab / (jnp.sum(rm_ab) + _EPS)
        term = p_ab * jnp.log((p_ab + _EPS) / (p_a * p_b + _EPS))
        return jnp.sum(term)

    o_ref[...] = jnp.full((8, 128), -(one_pair(0) + one_pair(1)), jnp.float32)


@functools.partial(jax.jit, static_argnames=("interpret",))
def _mi_loss(a, b, interpret=False):
    a3 = a.reshape(2, _NP, 1)
    b3 = b.reshape(2, _NP, 1)
    ac3 = a.reshape(2, 1, _NP)
    bc3 = b.reshape(2, 1, _NP)
    row_spec = pl.BlockSpec((1, _BR, 1), lambda p, r: (p, r, 0))
    col_spec = pl.BlockSpec((1, 1, _NP), lambda p, r: (p, 0, 0))
    rout_spec = pl.BlockSpec((1, _BR, 1), lambda p, r: (p, r, 0))
    cout_spec = pl.BlockSpec((1, 8, _NP), lambda p, r: (p, 0, 0))
    sums = pl.pallas_call(
        _rowsum_kernel,
        grid=(2, _M),
        in_specs=[row_spec, row_spec, col_spec, col_spec],
        out_specs=[rout_spec, rout_spec, rout_spec,
                   cout_spec, cout_spec, cout_spec],
        out_shape=[jax.ShapeDtypeStruct((2, _NP, 1), jnp.float32)] * 3
        + [jax.ShapeDtypeStruct((2, 8, _NP), jnp.float32)] * 3,
        compiler_params=pltpu.CompilerParams(
            dimension_semantics=("parallel", "arbitrary"),
        ),
        name="kde_rowsums",
        interpret=interpret,
    )(a3, b3, ac3, bc3)
    ra, rbv, rab = (s.reshape(2, _NP // 128, 128) for s in sums[:3])
    ca, cbv, cab = (s.reshape(2, 8, _NP // 128, 128) for s in sums[3:])
    out = pl.pallas_call(
        _mi_kernel,
        out_shape=jax.ShapeDtypeStruct((8, 128), jnp.float32),
        name="kde_mi_reduce",
        interpret=interpret,
    )(ra, rbv, rab, ca, cbv, cab)
    return out[0, 0]


def _prep(fused_img, source_img1, source_img2, idx1, idx2):
    f = fused_img.reshape(-1)
    s1 = source_img1.reshape(-1)
    s2 = source_img2.reshape(-1)
    scale = jnp.float32(1.0 / _SIGMA)
    a = jnp.stack([f[idx1], f[idx2]]) * scale
    b = jnp.stack([s1[idx1], s2[idx2]]) * scale
    pad = jnp.full((2, _NP - _SAMPLE), _PAD, jnp.float32)
    a = jnp.concatenate([a, pad], axis=1)
    b = jnp.concatenate([b, pad], axis=1)
    return a, b


def kernel(fused_img, source_img1, source_img2, idx1, idx2):
    a, b = _prep(fused_img, source_img1, source_img2, idx1, idx2)
    return _mi_loss(a, b)


# speedup vs baseline: 1.6512x; 1.6512x over previous
import functools

import jax
import jax.numpy as jnp
from jax.experimental import pallas as pl
from jax.experimental.pallas import tpu as pltpu

_SIGMA = 0.4
_SAMPLE = 10000
_EPS = 1e-10

_NP = 10240
_BR = 512
_CS = 16
_GRP = 16
_M = _NP // _BR
_PAD = 1e4
_C = -0.5 * 1.4426950408889634


def _rowsum_kernel(a_row_ref, b_row_ref, a_col_ref, b_col_ref,
                   oa_ref, ob_ref, oab_ref):
    bf = jnp.bfloat16
    cbf = bf(_C)
    arow = a_row_ref[0]
    brow = b_row_ref[0]
    def group(g, accs):
        acc_a, acc_b, acc_ab = accs
        base = g * _GRP * _CS
        for k in range(_GRP):
            off = pl.multiple_of(base + k * _CS, _CS)
            acol = a_col_ref[0, pl.ds(off, _CS), :]
            bcol = b_col_ref[0, pl.ds(off, _CS), :]
            aw = pltpu.repeat(acol, _BR // 128, axis=1)
            bw = pltpu.repeat(bcol, _BR // 128, axis=1)
            da = aw - arow
            db = bw - brow
            ka = jnp.exp2((da * da * cbf).astype(jnp.float32))
            kb = jnp.exp2((db * db * cbf).astype(jnp.float32))
            kab = ka * kb
            acc_a = acc_a + jnp.sum(ka.reshape(_CS // 8, 8, _BR), axis=0)
            acc_b = acc_b + jnp.sum(kb.reshape(_CS // 8, 8, _BR), axis=0)
            acc_ab = acc_ab + jnp.sum(kab.reshape(_CS // 8, 8, _BR), axis=0)
        return acc_a, acc_b, acc_ab

    zeros = jnp.zeros((8, _BR), jnp.float32)
    acc_a, acc_b, acc_ab = jax.lax.fori_loop(
        0, _NP // (_GRP * _CS), group, (zeros, zeros, zeros))
    oa_ref[...] = jnp.sum(acc_a, axis=0, keepdims=True).reshape(1, 1, _BR)
    ob_ref[...] = jnp.sum(acc_b, axis=0, keepdims=True).reshape(1, 1, _BR)
    oab_ref[...] = jnp.sum(acc_ab, axis=0, keepdims=True).reshape(1, 1, _BR)


def _mi_kernel(sa_ref, sb_ref, sab_ref, o_ref):
    rows = jax.lax.broadcasted_iota(jnp.int32, (_NP // 128, 128), 0)
    lanes = jax.lax.broadcasted_iota(jnp.int32, (_NP // 128, 128), 1)
    mask = (rows * 128 + lanes) < _SAMPLE
    inv_n = jnp.float32(1.0 / _SAMPLE)

    def one_pair(p):
        rm_a = jnp.where(mask, sa_ref[p] * inv_n, 0.0)
        rm_b = jnp.where(mask, sb_ref[p] * inv_n, 0.0)
        rm_ab = jnp.where(mask, sab_ref[p] * inv_n, 0.0)
        p_a = rm_a / (jnp.sum(rm_a) + _EPS)
        p_b = rm_b / (jnp.sum(rm_b) + _EPS)
        p_ab = rm_ab / (jnp.sum(rm_ab) + _EPS)
        term = p_ab * jnp.log((p_ab + _EPS) / (p_a * p_b + _EPS))
        return jnp.sum(term)

    o_ref[...] = jnp.full((8, 128), -(one_pair(0) + one_pair(1)), jnp.float32)


@functools.partial(jax.jit, static_argnames=("interpret",))
def _mi_loss(a, b, interpret=False):
    a2 = a.reshape(2, 1, _NP).astype(jnp.bfloat16)
    b2 = b.reshape(2, 1, _NP).astype(jnp.bfloat16)
    acolb = jnp.broadcast_to(a.reshape(2, _NP, 1), (2, _NP, 128)).astype(jnp.bfloat16)
    bcolb = jnp.broadcast_to(b.reshape(2, _NP, 1), (2, _NP, 128)).astype(jnp.bfloat16)
    row_spec = pl.BlockSpec((1, 1, _BR), lambda p, r: (p, 0, r))
    col_spec = pl.BlockSpec((1, _NP, 128), lambda p, r: (p, 0, 0))
    out_spec = pl.BlockSpec((1, 1, _BR), lambda p, r: (p, 0, r))
    sums = pl.pallas_call(
        _rowsum_kernel,
        grid=(2, _M),
        in_specs=[row_spec, row_spec, col_spec, col_spec],
        out_specs=[out_spec, out_spec, out_spec],
        out_shape=[jax.ShapeDtypeStruct((2, 1, _NP), jnp.float32)] * 3,
        compiler_params=pltpu.CompilerParams(
            dimension_semantics=("parallel", "arbitrary"),
        ),
        name="kde_rowsums",
        interpret=interpret,
    )(a2, b2, acolb, bcolb)
    sa, sb, sab = (s.reshape(2, _NP // 128, 128) for s in sums)
    out = pl.pallas_call(
        _mi_kernel,
        out_shape=jax.ShapeDtypeStruct((8, 128), jnp.float32),
        name="kde_mi_reduce",
        interpret=interpret,
    )(sa, sb, sab)
    return out[0, 0]


def _prep(fused_img, source_img1, source_img2, idx1, idx2):
    f = fused_img.reshape(-1)
    s1 = source_img1.reshape(-1)
    s2 = source_img2.reshape(-1)
    scale = jnp.float32(1.0 / _SIGMA)
    a = jnp.stack([f[idx1], f[idx2]]) * scale
    b = jnp.stack([s1[idx1], s2[idx2]]) * scale
    pad = jnp.full((2, _NP - _SAMPLE), _PAD, jnp.float32)
    a = jnp.concatenate([a, pad], axis=1)
    b = jnp.concatenate([b, pad], axis=1)
    return a, b


def kernel(fused_img, source_img1, source_img2, idx1, idx2):
    a, b = _prep(fused_img, source_img1, source_img2, idx1, idx2)
    return _mi_loss(a, b)


# speedup vs baseline: 1.7049x; 1.0325x over previous
import functools

import jax
import jax.numpy as jnp
from jax.experimental import pallas as pl
from jax.experimental.pallas import tpu as pltpu

_SIGMA = 0.4
_SAMPLE = 10000
_EPS = 1e-10

_NP = 10240
_BR = 512
_CS = 16
_GRP = 32
_M = _NP // _BR
_PAD = 1e4
_C = -0.5 * 1.4426950408889634


def _rowsum_kernel(a_row_ref, b_row_ref, a_col_ref, b_col_ref,
                   oa_ref, ob_ref, oab_ref):
    bf = jnp.bfloat16
    cbf = bf(_C)
    arow = a_row_ref[0]
    brow = b_row_ref[0]
    def group(g, accs):
        acc_a, acc_b, acc_ab = accs
        base = g * _GRP * _CS
        for k in range(_GRP):
            off = pl.multiple_of(base + k * _CS, _CS)
            acol = a_col_ref[0, pl.ds(off, _CS), :]
            bcol = b_col_ref[0, pl.ds(off, _CS), :]
            aw = pltpu.repeat(acol, _BR // 128, axis=1)
            bw = pltpu.repeat(bcol, _BR // 128, axis=1)
            da = aw - arow
            db = bw - brow
            ka = jnp.exp2((da * da * cbf).astype(jnp.float32))
            kb = jnp.exp2((db * db * cbf).astype(jnp.float32))
            kab = ka * kb
            acc_a = acc_a + jnp.sum(ka.reshape(_CS // 8, 8, _BR), axis=0)
            acc_b = acc_b + jnp.sum(kb.reshape(_CS // 8, 8, _BR), axis=0)
            acc_ab = acc_ab + jnp.sum(kab.reshape(_CS // 8, 8, _BR), axis=0)
        return acc_a, acc_b, acc_ab

    zeros = jnp.zeros((8, _BR), jnp.float32)
    acc_a, acc_b, acc_ab = jax.lax.fori_loop(
        0, _NP // (_GRP * _CS), group, (zeros, zeros, zeros))
    oa_ref[...] = jnp.sum(acc_a, axis=0, keepdims=True).reshape(1, 1, _BR)
    ob_ref[...] = jnp.sum(acc_b, axis=0, keepdims=True).reshape(1, 1, _BR)
    oab_ref[...] = jnp.sum(acc_ab, axis=0, keepdims=True).reshape(1, 1, _BR)


def _mi_kernel(sa_ref, sb_ref, sab_ref, o_ref):
    rows = jax.lax.broadcasted_iota(jnp.int32, (_NP // 128, 128), 0)
    lanes = jax.lax.broadcasted_iota(jnp.int32, (_NP // 128, 128), 1)
    mask = (rows * 128 + lanes) < _SAMPLE
    inv_n = jnp.float32(1.0 / _SAMPLE)

    def one_pair(p):
        rm_a = jnp.where(mask, sa_ref[p] * inv_n, 0.0)
        rm_b = jnp.where(mask, sb_ref[p] * inv_n, 0.0)
        rm_ab = jnp.where(mask, sab_ref[p] * inv_n, 0.0)
        p_a = rm_a / (jnp.sum(rm_a) + _EPS)
        p_b = rm_b / (jnp.sum(rm_b) + _EPS)
        p_ab = rm_ab / (jnp.sum(rm_ab) + _EPS)
        term = p_ab * jnp.log((p_ab + _EPS) / (p_a * p_b + _EPS))
        return jnp.sum(term)

    o_ref[...] = jnp.full((8, 128), -(one_pair(0) + one_pair(1)), jnp.float32)


@functools.partial(jax.jit, static_argnames=("interpret",))
def _mi_loss(a, b, interpret=False):
    a2 = a.reshape(2, 1, _NP).astype(jnp.bfloat16)
    b2 = b.reshape(2, 1, _NP).astype(jnp.bfloat16)
    acolb = jnp.broadcast_to(a.reshape(2, _NP, 1), (2, _NP, 128)).astype(jnp.bfloat16)
    bcolb = jnp.broadcast_to(b.reshape(2, _NP, 1), (2, _NP, 128)).astype(jnp.bfloat16)
    row_spec = pl.BlockSpec((1, 1, _BR), lambda p, r: (p, 0, r))
    col_spec = pl.BlockSpec((1, _NP, 128), lambda p, r: (p, 0, 0))
    out_spec = pl.BlockSpec((1, 1, _BR), lambda p, r: (p, 0, r))
    sums = pl.pallas_call(
        _rowsum_kernel,
        grid=(2, _M),
        in_specs=[row_spec, row_spec, col_spec, col_spec],
        out_specs=[out_spec, out_spec, out_spec],
        out_shape=[jax.ShapeDtypeStruct((2, 1, _NP), jnp.float32)] * 3,
        compiler_params=pltpu.CompilerParams(
            dimension_semantics=("parallel", "arbitrary"),
        ),
        name="kde_rowsums",
        interpret=interpret,
    )(a2, b2, acolb, bcolb)
    sa, sb, sab = (s.reshape(2, _NP // 128, 128) for s in sums)
    out = pl.pallas_call(
        _mi_kernel,
        out_shape=jax.ShapeDtypeStruct((8, 128), jnp.float32),
        name="kde_mi_reduce",
        interpret=interpret,
    )(sa, sb, sab)
    return out[0, 0]


def _prep(fused_img, source_img1, source_img2, idx1, idx2):
    f = fused_img.reshape(-1)
    s1 = source_img1.reshape(-1)
    s2 = source_img2.reshape(-1)
    scale = jnp.float32(1.0 / _SIGMA)
    a = jnp.stack([f[idx1], f[idx2]]) * scale
    b = jnp.stack([s1[idx1], s2[idx2]]) * scale
    pad = jnp.full((2, _NP - _SAMPLE), _PAD, jnp.float32)
    a = jnp.concatenate([a, pad], axis=1)
    b = jnp.concatenate([b, pad], axis=1)
    return a, b


def kernel(fused_img, source_img1, source_img2, idx1, idx2):
    a, b = _prep(fused_img, source_img1, source_img2, idx1, idx2)
    return _mi_loss(a, b)
